# baseline (speedup 1.0000x reference)
import functools

import jax
import jax.numpy as jnp
from jax import lax
from jax.experimental import pallas as pl
from jax.experimental.pallas import tpu as pltpu
from jax.experimental.pallas import tpu_sc as plsc

_N = 10000
_D = 128
_CHUNK = 128
_CPT = 80
_EPAD = 32 * _CPT * _CHUNK
_NCHUNKS = _EPAD // _CHUNK
_NP = 10240
_HNP = _NP // 2
_ACR = _NP
_ZPS = _ACR // 16
_OPS = _NP // 16
_DEGW = 16

_mesh = plsc.VectorSubcoreMesh(core_axis_name="c", subcore_axis_name="s")


def _fill(ref, nrows, ncol16, val):
    v = jnp.full((16,), val, jnp.float32)

    def body(r, carry):
        for cc in range(ncol16):
            ref[r, pl.ds(cc * 16, 16)] = v
        return carry

    lax.fori_loop(0, nrows, body, 0)


def _zero_acc(acc, zbuf, s, zps=_ZPS):
    base = s * zps
    for k in range(zps // _CHUNK):
        pltpu.sync_copy(zbuf, acc.at[pl.ds(base + k * _CHUNK, _CHUNK)])
    rem = zps % _CHUNK
    if rem:
        pltpu.sync_copy(zbuf.at[pl.ds(0, rem)],
                        acc.at[pl.ds(base + (zps // _CHUNK) * _CHUNK, rem)])


_NBUF = 2
_IQ = 40

_AGG2_SCRATCH = [
    pltpu.VMEM((_IQ, _CHUNK), jnp.int32),
    pltpu.VMEM((_IQ, _CHUNK), jnp.int32),
    pltpu.VMEM((32, 4 * _D), jnp.float32),
    pltpu.VMEM((32, 4 * _D), jnp.float32),
    pltpu.VMEM_SHARED((_ACR, _D), jnp.float32),
    pltpu.SemaphoreType.DMA,
    pltpu.SemaphoreType.DMA,
]


def _agg2_body(featx, feath, featw, srcp, dstp, out,
               src_b, dst_b, r0, r1, acc, g0, g1):
    c = lax.axis_index("c")
    s = lax.axis_index("s")
    wid = c * 16 + s
    rows = [r0, r1]
    sems = [g0, g1]
    for fi, feat in enumerate([featx, feath]):
        plsc.subcore_barrier()

        def round_body(J, carry):
            for b in range(_NBUF):
                j = J * _NBUF + b
                idx = src_b.at[j % _IQ, pl.ds((j // _IQ) * 32, 32)]
                pltpu.make_async_copy(featw.at[idx], rows[b], sems[b]).wait()
                jn = jnp.minimum(j + _NBUF, 4 * _IQ - 1)
                idxn = src_b.at[jn % _IQ, pl.ds((jn // _IQ) * 32, 32)]
                pltpu.async_copy(featw.at[idxn], rows[b], sems[b])
            return carry

        for q in range(_CPT // _IQ):
            pltpu.sync_copy(srcp.at[pl.ds(wid * _CPT + q * _IQ, _IQ)], src_b)
            pltpu.sync_copy(dstp.at[pl.ds(wid * _CPT + q * _IQ, _IQ)], dst_b)
            for b in range(_NBUF):
                pltpu.async_copy(featw.at[src_b.at[0, pl.ds(0, 32)]],
                                 rows[b], sems[b])
            lax.fori_loop(0, 4 * _IQ // _NBUF, round_body, 0)
            for b in range(_NBUF):
                pltpu.make_async_copy(featw.at[src_b.at[0, pl.ds(0, 32)]],
                                      rows[b], sems[b]).wait()
        plsc.subcore_barrier()
        pltpu.sync_copy(acc.at[pl.ds(s * _OPS, _OPS)],
                        out.at[fi, c, pl.ds(s * _OPS, _OPS)])
        plsc.subcore_barrier()


_agg2 = pl.kernel(
    _agg2_body,
    out_type=jax.ShapeDtypeStruct((2, 2, _NP, _D), jnp.float32),
    mesh=_mesh,
    scratch_types=_AGG2_SCRATCH,
)

_DACR = _HNP + 256

_DEG_SCRATCH = [
    pltpu.VMEM((2 * _CPT, _CHUNK), jnp.int32),
    pltpu.VMEM((_CHUNK, _D), jnp.float32),
    pltpu.VMEM((_CHUNK, _D), jnp.float32),
    pltpu.VMEM_SHARED((_DACR, _D), jnp.float32),
]


def _deg_body(dsts, outdeg, dst_b, zdeg, ones_b, dacc):
    c = lax.axis_index("c")
    s = lax.axis_index("s")
    _fill(zdeg, _CHUNK, _D // 16, 0.0)
    _fill(ones_b, _CHUNK, _D // 16, 1.0)
    pltpu.sync_copy(dsts.at[c, pl.ds(s * 2 * _CPT, 2 * _CPT)], dst_b)
    base = s * (_DACR // 16)
    for k in range(_DACR // 16 // _CHUNK):
        pltpu.sync_copy(zdeg, dacc.at[pl.ds(base + k * _CHUNK, _CHUNK)])
    pltpu.sync_copy(zdeg.at[pl.ds(0, _DACR // 16 - 2 * _CHUNK)],
                    dacc.at[pl.ds(base + 2 * _CHUNK, _DACR // 16 - 2 * _CHUNK)])
    plsc.subcore_barrier()

    def chunk_body(j, carry):
        pltpu.sync_copy(ones_b, dacc.at[dst_b.at[j]], add=True)
        return carry

    lax.fori_loop(0, 2 * _CPT, chunk_body, 0)
    plsc.subcore_barrier()
    pltpu.sync_copy(dacc.at[pl.ds(s * (_HNP // 16), _HNP // 16)],
                    outdeg.at[pl.ds(c * _HNP + s * (_HNP // 16), _HNP // 16)])


_deg = pl.kernel(
    _deg_body,
    out_type=jax.ShapeDtypeStruct((_NP, _D), jnp.float32),
    mesh=_mesh,
    scratch_types=_DEG_SCRATCH,
)


def _gru_body(px, ph, dp, h, w, b, o):
    inv = 1.0 / jnp.maximum(dp[:, 0:1], 1.0)
    mx = (px[0] + px[1]) * inv
    mh = (ph[0] + ph[1]) * inv
    g = (jnp.dot(mx, w[0], preferred_element_type=jnp.float32)
         + jnp.dot(mh, w[1], preferred_element_type=jnp.float32)
         + b[...])
    u = jax.nn.sigmoid(g[:, :_D])
    cand = jnp.tanh(g[:, _D:])
    o[...] = u * h[...] + (1.0 - u) * cand


_ROWS_BLK = 1024

_gru = pl.pallas_call(
    _gru_body,
    grid=(_NP // _ROWS_BLK,),
    in_specs=[
        pl.BlockSpec((2, _ROWS_BLK, _D), lambda i: (0, i, 0)),
        pl.BlockSpec((2, _ROWS_BLK, _D), lambda i: (0, i, 0)),
        pl.BlockSpec((_ROWS_BLK, _D), lambda i: (i, 0)),
        pl.BlockSpec((_ROWS_BLK, _D), lambda i: (i, 0)),
        pl.BlockSpec((2, _D, 2 * _D), lambda i: (0, 0, 0)),
        pl.BlockSpec((1, 2 * _D), lambda i: (0, 0)),
    ],
    out_specs=pl.BlockSpec((_ROWS_BLK, _D), lambda i: (i, 0)),
    out_shape=jax.ShapeDtypeStruct((_NP, _D), jnp.float32),
)


def kernel(x, edge_index, hidden_states, Ws, bs):
    seq_len = x.shape[0]
    npad = _EPAD - edge_index.shape[1]
    src = jnp.concatenate([edge_index[0], jnp.zeros((npad,), jnp.int32)])
    dst = jnp.concatenate([edge_index[1], jnp.full((npad,), _N, jnp.int32)])
    dst0 = jnp.where(dst < _HNP, dst, _HNP)
    dst1 = jnp.where(dst >= _HNP, dst - _HNP, _HNP)
    srcp = src.reshape(_NCHUNKS, _CHUNK)
    dstp = dst.reshape(_NCHUNKS, _CHUNK)
    dsts = jnp.stack([dst0, dst1]).reshape(2, _NCHUNKS, _CHUNK)

    xp = jnp.pad(x, ((0, 0), (0, _NP - _N), (0, 0)))
    xw = jnp.transpose(xp, (1, 0, 2)).reshape(_NP, 4 * _D)
    outdeg = _deg(dsts)

    W = Ws[0]
    b = bs[0]
    Wc = jnp.stack([jnp.concatenate([W[2], W[4]], axis=1),
                    jnp.concatenate([W[3], W[5]], axis=1)])
    bc = jnp.concatenate([b[2] + b[3], b[4] + b[5]]).reshape(1, 2 * _D)

    h = jnp.pad(hidden_states[0], ((0, _NP - _N), (0, 0)))
    for t in range(seq_len):
        p = _agg2(xp[t], h, xw, srcp, dstp)
        h = _gru(p[0], p[1], outdeg, h, Wc, bc)
    return (x, h[:_N][None])


# speedup vs baseline: 1.2215x; 1.2215x over previous
import functools

import jax
import jax.numpy as jnp
from jax import lax
from jax.experimental import pallas as pl
from jax.experimental.pallas import tpu as pltpu
from jax.experimental.pallas import tpu_sc as plsc

_N = 10000
_D = 128
_CHUNK = 128
_CPT = 80
_EPAD = 32 * _CPT * _CHUNK
_NCHUNKS = _EPAD // _CHUNK
_NP = 10240
_HNP = _NP // 2
_ACR = _NP
_ZPS = _ACR // 16
_OPS = _NP // 16
_DEGW = 16

_mesh = plsc.VectorSubcoreMesh(core_axis_name="c", subcore_axis_name="s")


def _fill(ref, nrows, ncol16, val):
    v = jnp.full((16,), val, jnp.float32)

    def body(r, carry):
        for cc in range(ncol16):
            ref[r, pl.ds(cc * 16, 16)] = v
        return carry

    lax.fori_loop(0, nrows, body, 0)


def _zero_acc2(acc, z0, z1, s, zps=None):
    zps = zps if zps is not None else _ZPS
    base = s * zps
    for k in range(zps // _CHUNK):
        pltpu.sync_copy(z0, acc.at[pl.ds(base + k * _CHUNK, _HC)])
        pltpu.sync_copy(z1, acc.at[pl.ds(base + k * _CHUNK + _HC, _HC)])
    rem = zps % _CHUNK
    if rem:
        off = base + (zps // _CHUNK) * _CHUNK
        pltpu.sync_copy(z0.at[pl.ds(0, min(rem, _HC))],
                        acc.at[pl.ds(off, min(rem, _HC))])
        if rem > _HC:
            pltpu.sync_copy(z1.at[pl.ds(0, rem - _HC)],
                            acc.at[pl.ds(off + _HC, rem - _HC)])


def _zero_acc(acc, zbuf, s, zps=_ZPS):
    base = s * zps
    for k in range(zps // _CHUNK):
        pltpu.sync_copy(zbuf, acc.at[pl.ds(base + k * _CHUNK, _CHUNK)])
    rem = zps % _CHUNK
    if rem:
        pltpu.sync_copy(zbuf.at[pl.ds(0, rem)],
                        acc.at[pl.ds(base + (zps // _CHUNK) * _CHUNK, rem)])


_NBUF = 4
_HC = 64
_IQ = 40

_AGG2_SCRATCH = [
    pltpu.VMEM((_IQ, _CHUNK), jnp.int32),
    pltpu.VMEM((_IQ, _CHUNK), jnp.int32),
    pltpu.VMEM((_HC, _D), jnp.float32),
    pltpu.VMEM((_HC, _D), jnp.float32),
    pltpu.VMEM((_HC, _D), jnp.float32),
    pltpu.VMEM((_HC, _D), jnp.float32),
    pltpu.VMEM_SHARED((_ACR, _D), jnp.float32),
    pltpu.SemaphoreType.DMA,
    pltpu.SemaphoreType.DMA,
    pltpu.SemaphoreType.DMA,
    pltpu.SemaphoreType.DMA,
]


def _agg2_body(featx, feath, srcp, dstp, out,
               src_b, dst_b, r0, r1, r2, r3, acc, g0, g1, g2, g3):
    c = lax.axis_index("c")
    s = lax.axis_index("s")
    wid = c * 16 + s
    rows = [r0, r1, r2, r3]
    sems = [g0, g1, g2, g3]
    for fi, feat in enumerate([featx, feath]):
        for zb in (r0, r1):
            _fill(zb, _HC, _D // 16, 0.0)
        _zero_acc2(acc, r0, r1, s)
        plsc.subcore_barrier()

        nhc = 2 * _IQ

        def round_body(J, carry):
            for b in range(_NBUF):
                j = J * _NBUF + b
                idx = src_b.at[j // 2, pl.ds((j % 2) * _HC, _HC)]
                didx = dst_b.at[j // 2, pl.ds((j % 2) * _HC, _HC)]
                pltpu.make_async_copy(feat.at[idx], rows[b], sems[b]).wait()
                pltpu.sync_copy(rows[b], acc.at[didx], add=True)
                jn = jnp.minimum(j + _NBUF, nhc - 1)
                idxn = src_b.at[jn // 2, pl.ds((jn % 2) * _HC, _HC)]
                pltpu.async_copy(feat.at[idxn], rows[b], sems[b])
            return carry

        for q in range(_CPT // _IQ):
            pltpu.sync_copy(srcp.at[pl.ds(wid * _CPT + q * _IQ, _IQ)], src_b)
            pltpu.sync_copy(dstp.at[pl.ds(wid * _CPT + q * _IQ, _IQ)], dst_b)
            for b in range(_NBUF):
                idx = src_b.at[b // 2, pl.ds((b % 2) * _HC, _HC)]
                pltpu.async_copy(feat.at[idx], rows[b], sems[b])
            lax.fori_loop(0, nhc // _NBUF, round_body, 0)
            for b in range(_NBUF):
                pltpu.make_async_copy(feat.at[src_b.at[0, pl.ds(0, _HC)]],
                                      rows[b], sems[b]).wait()
        plsc.subcore_barrier()
        pltpu.sync_copy(acc.at[pl.ds(s * _OPS, _OPS)],
                        out.at[fi, c, pl.ds(s * _OPS, _OPS)])
        plsc.subcore_barrier()


_agg2 = pl.kernel(
    _agg2_body,
    out_type=jax.ShapeDtypeStruct((2, 2, _NP, _D), jnp.float32),
    mesh=_mesh,
    scratch_types=_AGG2_SCRATCH,
)

_DACR = _HNP + 256

_DEG_SCRATCH = [
    pltpu.VMEM((2 * _CPT, _CHUNK), jnp.int32),
    pltpu.VMEM((_CHUNK, _D), jnp.float32),
    pltpu.VMEM((_CHUNK, _D), jnp.float32),
    pltpu.VMEM_SHARED((_DACR, _D), jnp.float32),
]


def _deg_body(dsts, outdeg, dst_b, zdeg, ones_b, dacc):
    c = lax.axis_index("c")
    s = lax.axis_index("s")
    _fill(zdeg, _CHUNK, _D // 16, 0.0)
    _fill(ones_b, _CHUNK, _D // 16, 1.0)
    pltpu.sync_copy(dsts.at[c, pl.ds(s * 2 * _CPT, 2 * _CPT)], dst_b)
    base = s * (_DACR // 16)
    for k in range(_DACR // 16 // _CHUNK):
        pltpu.sync_copy(zdeg, dacc.at[pl.ds(base + k * _CHUNK, _CHUNK)])
    pltpu.sync_copy(zdeg.at[pl.ds(0, _DACR // 16 - 2 * _CHUNK)],
                    dacc.at[pl.ds(base + 2 * _CHUNK, _DACR // 16 - 2 * _CHUNK)])
    plsc.subcore_barrier()

    def chunk_body(j, carry):
        pltpu.sync_copy(ones_b, dacc.at[dst_b.at[j]], add=True)
        return carry

    lax.fori_loop(0, 2 * _CPT, chunk_body, 0)
    plsc.subcore_barrier()
    pltpu.sync_copy(dacc.at[pl.ds(s * (_HNP // 16), _HNP // 16)],
                    outdeg.at[pl.ds(c * _HNP + s * (_HNP // 16), _HNP // 16)])


_deg = pl.kernel(
    _deg_body,
    out_type=jax.ShapeDtypeStruct((_NP, _D), jnp.float32),
    mesh=_mesh,
    scratch_types=_DEG_SCRATCH,
)


def _gru_body(px, ph, dp, h, w, b, o):
    inv = 1.0 / jnp.maximum(dp[:, 0:1], 1.0)
    mx = (px[0] + px[1]) * inv
    mh = (ph[0] + ph[1]) * inv
    g = (jnp.dot(mx, w[0], preferred_element_type=jnp.float32)
         + jnp.dot(mh, w[1], preferred_element_type=jnp.float32)
         + b[...])
    u = jax.nn.sigmoid(g[:, :_D])
    cand = jnp.tanh(g[:, _D:])
    o[...] = u * h[...] + (1.0 - u) * cand


_ROWS_BLK = 1024

_gru = pl.pallas_call(
    _gru_body,
    grid=(_NP // _ROWS_BLK,),
    in_specs=[
        pl.BlockSpec((2, _ROWS_BLK, _D), lambda i: (0, i, 0)),
        pl.BlockSpec((2, _ROWS_BLK, _D), lambda i: (0, i, 0)),
        pl.BlockSpec((_ROWS_BLK, _D), lambda i: (i, 0)),
        pl.BlockSpec((_ROWS_BLK, _D), lambda i: (i, 0)),
        pl.BlockSpec((2, _D, 2 * _D), lambda i: (0, 0, 0)),
        pl.BlockSpec((1, 2 * _D), lambda i: (0, 0)),
    ],
    out_specs=pl.BlockSpec((_ROWS_BLK, _D), lambda i: (i, 0)),
    out_shape=jax.ShapeDtypeStruct((_NP, _D), jnp.float32),
)


def kernel(x, edge_index, hidden_states, Ws, bs):
    seq_len = x.shape[0]
    npad = _EPAD - edge_index.shape[1]
    src = jnp.concatenate([edge_index[0], jnp.zeros((npad,), jnp.int32)])
    dst = jnp.concatenate([edge_index[1], jnp.full((npad,), _N, jnp.int32)])
    dst0 = jnp.where(dst < _HNP, dst, _HNP)
    dst1 = jnp.where(dst >= _HNP, dst - _HNP, _HNP)
    srcp = src.reshape(_NCHUNKS, _CHUNK)
    dstp = dst.reshape(_NCHUNKS, _CHUNK)
    dsts = jnp.stack([dst0, dst1]).reshape(2, _NCHUNKS, _CHUNK)

    xp = jnp.pad(x, ((0, 0), (0, _NP - _N), (0, 0)))
    outdeg = _deg(dsts)

    W = Ws[0]
    b = bs[0]
    Wc = jnp.stack([jnp.concatenate([W[2], W[4]], axis=1),
                    jnp.concatenate([W[3], W[5]], axis=1)])
    bc = jnp.concatenate([b[2] + b[3], b[4] + b[5]]).reshape(1, 2 * _D)

    h = jnp.pad(hidden_states[0], ((0, _NP - _N), (0, 0)))
    for t in range(seq_len):
        p = _agg2(xp[t], h, srcp, dstp)
        h = _gru(p[0], p[1], outdeg, h, Wc, bc)
    return (x, h[:_N][None])
